# baseline (speedup 1.0000x reference)
import functools

import numpy as np
import jax
import jax.numpy as jnp
from jax import lax
from jax.experimental import pallas as pl
from jax.experimental.pallas import tpu as pltpu
from jax.experimental.pallas import tpu_sc as plsc

_D_MODEL = 64
_CONTEXT_LEN = 200


def _sinusoidal_table(d_model: int, context_len: int) -> np.ndarray:
    pos = np.arange(context_len, dtype=np.float32)[:, None]
    i = np.arange(d_model, dtype=np.float32)[None, :]
    angle_rates = 1.0 / np.power(10000.0, (2.0 * np.floor(i / 2.0)) / float(d_model))
    angles = pos * angle_rates
    table = np.zeros((context_len, d_model), dtype=np.float32)
    table[:, 0::2] = np.sin(angles[:, 0::2])
    table[:, 1::2] = np.cos(angles[:, 1::2])
    return table


def kernel(text_batch, embedding_matrix):
    batch, seq_len = text_batch.shape
    vocab, d_model = embedding_matrix.shape
    assert seq_len == _CONTEXT_LEN and d_model == _D_MODEL

    n_tokens = batch * seq_len
    flat_idx = text_batch.reshape(n_tokens)

    info = plsc.get_sparse_core_info()
    num_workers = info.num_cores * info.num_subcores
    per_worker = n_tokens // num_workers
    assert per_worker * num_workers == n_tokens
    chunk = seq_len
    n_chunks = per_worker // chunk
    assert n_chunks * chunk == per_worker and n_chunks % 2 == 0

    pos_table = jnp.asarray(_sinusoidal_table(d_model, seq_len))

    mesh = plsc.VectorSubcoreMesh(core_axis_name="c", subcore_axis_name="s")

    @functools.partial(
        pl.kernel,
        mesh=mesh,
        out_type=jax.ShapeDtypeStruct((n_tokens, d_model), jnp.float32),
        scratch_types=[
            pltpu.VMEM((per_worker,), jnp.int32),
            pltpu.VMEM((seq_len, d_model), jnp.float32),
            pltpu.VMEM((chunk, d_model), jnp.float32),
            pltpu.VMEM((chunk, d_model), jnp.float32),
            pltpu.VMEM((chunk, d_model), jnp.float32),
            pltpu.VMEM((chunk, d_model), jnp.float32),
            pltpu.SemaphoreType.DMA,
            pltpu.SemaphoreType.DMA,
            pltpu.SemaphoreType.DMA,
            pltpu.SemaphoreType.DMA,
        ],
        compiler_params=pltpu.CompilerParams(use_tc_tiling_on_sc=False),
    )
    def _emb_kernel(idx_hbm, table_hbm, pos_hbm, out_hbm, idx_v, pos_v,
                    gb0, gb1, ob0, ob1, sg0, sg1, sw0, sw1):
        gb = (gb0, gb1)
        ob = (ob0, ob1)
        sg = (sg0, sg1)
        sw = (sw0, sw1)

        wid = lax.axis_index("s") * info.num_cores + lax.axis_index("c")
        base = wid * per_worker
        pltpu.sync_copy(idx_hbm.at[pl.ds(base, per_worker)], idx_v)
        pltpu.sync_copy(pos_hbm, pos_v)

        def gather_rows(j, b):
            pltpu.async_copy(
                table_hbm.at[idx_v.at[pl.ds(j * chunk, chunk)]], gb[b], sg[b])

        for b in range(2):
            gather_rows(b, b)

        def pair_body(i, carry):
            j0 = i * 2
            for b in range(2):
                j = j0 + b
                @pl.when(j0 >= 2)
                def _wait_prev_write():
                    pltpu.make_async_copy(
                        ob[b], out_hbm.at[pl.ds(0, chunk)], sw[b]).wait()

                pltpu.make_async_copy(
                    table_hbm.at[idx_v.at[pl.ds(j * chunk, chunk)]],
                    gb[b], sg[b]).wait()

                @pl.when(j + 2 < n_chunks)
                def _next_gather():
                    gather_rows(j + 2, b)

                def row_body(r, c2):
                    for c in range(d_model // 16):
                        sl = pl.ds(c * 16, 16)
                        ob[b][r, sl] = gb[b][r, sl] + pos_v[r, sl]
                    return c2

                lax.fori_loop(0, chunk, row_body, 0, unroll=2)

                pltpu.async_copy(ob[b], out_hbm.at[pl.ds(base + j * chunk, chunk)], sw[b])
            return carry

        lax.fori_loop(0, n_chunks // 2, pair_body, 0)

        for b in range(2):
            pltpu.make_async_copy(ob[b], out_hbm.at[pl.ds(0, chunk)], sw[b]).wait()

    out = _emb_kernel(flat_idx, embedding_matrix, pos_table)
    return out.reshape(batch, seq_len, d_model)


# speedup vs baseline: 1.0000x; 1.0000x over previous
import functools

import numpy as np
import jax
import jax.numpy as jnp
from jax import lax
from jax.experimental import pallas as pl
from jax.experimental.pallas import tpu as pltpu
from jax.experimental.pallas import tpu_sc as plsc

_D_MODEL = 64
_CONTEXT_LEN = 200


def _sinusoidal_table(d_model: int, context_len: int) -> np.ndarray:
    pos = np.arange(context_len, dtype=np.float32)[:, None]
    i = np.arange(d_model, dtype=np.float32)[None, :]
    angle_rates = 1.0 / np.power(10000.0, (2.0 * np.floor(i / 2.0)) / float(d_model))
    angles = pos * angle_rates
    table = np.zeros((context_len, d_model), dtype=np.float32)
    table[:, 0::2] = np.sin(angles[:, 0::2])
    table[:, 1::2] = np.cos(angles[:, 1::2])
    return table


def kernel(text_batch, embedding_matrix):
    batch, seq_len = text_batch.shape
    vocab, d_model = embedding_matrix.shape
    assert seq_len == _CONTEXT_LEN and d_model == _D_MODEL

    n_tokens = batch * seq_len
    flat_idx = text_batch.reshape(n_tokens)

    info = plsc.get_sparse_core_info()
    num_workers = info.num_cores * info.num_subcores
    per_worker = n_tokens // num_workers
    assert per_worker * num_workers == n_tokens
    chunk = seq_len
    n_chunks = per_worker // chunk
    assert n_chunks * chunk == per_worker and n_chunks % 2 == 0

    pos_table = jnp.asarray(_sinusoidal_table(d_model, seq_len))

    mesh = plsc.VectorSubcoreMesh(core_axis_name="c", subcore_axis_name="s")

    @functools.partial(
        pl.kernel,
        mesh=mesh,
        out_type=jax.ShapeDtypeStruct((n_tokens, d_model), jnp.float32),
        scratch_types=[
            pltpu.VMEM((per_worker,), jnp.int32),
            pltpu.VMEM((seq_len, d_model), jnp.float32),
            pltpu.VMEM((chunk, d_model), jnp.float32),
            pltpu.VMEM((chunk, d_model), jnp.float32),
            pltpu.VMEM((chunk, d_model), jnp.float32),
            pltpu.VMEM((chunk, d_model), jnp.float32),
            pltpu.SemaphoreType.DMA,
            pltpu.SemaphoreType.DMA,
            pltpu.SemaphoreType.DMA,
            pltpu.SemaphoreType.DMA,
        ],
        compiler_params=pltpu.CompilerParams(use_tc_tiling_on_sc=False),
    )
    def _emb_kernel(idx_hbm, table_hbm, pos_hbm, out_hbm, idx_v, pos_v,
                    gb0, gb1, ob0, ob1, sg0, sg1, sw0, sw1):
        gb = (gb0, gb1)
        ob = (ob0, ob1)
        sg = (sg0, sg1)
        sw = (sw0, sw1)

        wid = lax.axis_index("s") * info.num_cores + lax.axis_index("c")
        base = wid * per_worker
        pltpu.sync_copy(idx_hbm.at[pl.ds(base, per_worker)], idx_v)
        pltpu.sync_copy(pos_hbm, pos_v)

        def gather_rows(j, b):
            pltpu.async_copy(
                table_hbm.at[idx_v.at[pl.ds(j * chunk, chunk)]], gb[b], sg[b])

        for b in range(2):
            gather_rows(b, b)

        def pair_body(i, carry):
            j0 = i * 2
            for b in range(2):
                j = j0 + b
                @pl.when(j0 >= 2)
                def _wait_prev_write():
                    pltpu.make_async_copy(
                        ob[b], out_hbm.at[pl.ds(0, chunk)], sw[b]).wait()

                pltpu.make_async_copy(
                    table_hbm.at[idx_v.at[pl.ds(j * chunk, chunk)]],
                    gb[b], sg[b]).wait()

                def row_body(r, c2):
                    for c in range(d_model // 16):
                        sl = pl.ds(c * 16, 16)
                        ob[b][r, sl] = gb[b][r, sl] + pos_v[r, sl]
                    return c2

                lax.fori_loop(0, chunk, row_body, 0, unroll=2)

                @pl.when(j + 2 < n_chunks)
                def _next_gather():
                    gather_rows(j + 2, b)

                pltpu.async_copy(ob[b], out_hbm.at[pl.ds(base + j * chunk, chunk)], sw[b])
            return carry

        lax.fori_loop(0, n_chunks // 2, pair_body, 0)

        for b in range(2):
            pltpu.make_async_copy(ob[b], out_hbm.at[pl.ds(0, chunk)], sw[b]).wait()

    out = _emb_kernel(flat_idx, embedding_matrix, pos_table)
    return out.reshape(batch, seq_len, d_model)


# speedup vs baseline: 2.0980x; 2.0979x over previous
import functools

import numpy as np
import jax
import jax.numpy as jnp
from jax import lax
from jax.experimental import pallas as pl
from jax.experimental.pallas import tpu as pltpu
from jax.experimental.pallas import tpu_sc as plsc

_D_MODEL = 64
_CONTEXT_LEN = 200
_NBUF = 4


def _sinusoidal_table(d_model: int, context_len: int) -> np.ndarray:
    pos = np.arange(context_len, dtype=np.float32)[:, None]
    i = np.arange(d_model, dtype=np.float32)[None, :]
    angle_rates = 1.0 / np.power(10000.0, (2.0 * np.floor(i / 2.0)) / float(d_model))
    angles = pos * angle_rates
    table = np.zeros((context_len, d_model), dtype=np.float32)
    table[:, 0::2] = np.sin(angles[:, 0::2])
    table[:, 1::2] = np.cos(angles[:, 1::2])
    return table


def kernel(text_batch, embedding_matrix):
    batch, seq_len = text_batch.shape
    vocab, d_model = embedding_matrix.shape
    assert seq_len == _CONTEXT_LEN and d_model == _D_MODEL

    n_tokens = batch * seq_len
    flat_idx = text_batch.reshape(n_tokens)
    table3 = embedding_matrix.reshape(vocab, 1, d_model)

    info = plsc.get_sparse_core_info()
    num_workers = info.num_cores * info.num_subcores
    per_worker = n_tokens // num_workers
    assert per_worker * num_workers == n_tokens
    chunk = seq_len
    n_chunks = per_worker // chunk
    assert n_chunks * chunk == per_worker and n_chunks % _NBUF == 0

    pos_flat = jnp.asarray(_sinusoidal_table(d_model, seq_len).reshape(-1))

    mesh = plsc.VectorSubcoreMesh(core_axis_name="c", subcore_axis_name="s")

    @functools.partial(
        pl.kernel,
        mesh=mesh,
        out_type=jax.ShapeDtypeStruct((n_tokens, 1, d_model), jnp.float32),
        scratch_types=[
            [pltpu.VMEM((chunk,), jnp.int32) for _ in range(_NBUF)],
            [pltpu.VMEM((chunk, 1, d_model), jnp.float32) for _ in range(_NBUF)],
            pltpu.VMEM((seq_len * d_model,), jnp.float32),
            [pltpu.SemaphoreType.DMA for _ in range(_NBUF)],
            [pltpu.SemaphoreType.DMA for _ in range(_NBUF)],
            [pltpu.SemaphoreType.DMA for _ in range(_NBUF)],
        ],
    )
    def _emb_kernel(idx_hbm, table_hbm, pos_hbm, out_hbm,
                    idx_c, gb, pos_v, s_ix, s_g, s_w):
        wid = lax.axis_index("s") * info.num_cores + lax.axis_index("c")
        base = wid * per_worker
        pltpu.sync_copy(pos_hbm, pos_v)

        def idx_copy(j, k):
            return pltpu.make_async_copy(
                idx_hbm.at[pl.ds(base + j * chunk, chunk)], idx_c[k], s_ix[k])

        def gather_copy(k):
            return pltpu.make_async_copy(table_hbm.at[idx_c[k]], gb[k], s_g[k])

        def write_copy(j, k):
            return pltpu.make_async_copy(
                gb[k], out_hbm.at[pl.ds(base + j * chunk, chunk)], s_w[k])

        for k in range(_NBUF):
            idx_copy(k, k).start()
        for k in range(2):
            idx_copy(k, k).wait()
            gather_copy(k).start()

        def quad_body(i, carry):
            j0 = i * _NBUF
            for b in range(_NBUF):
                j = j0 + b
                bn = (b + 2) % _NBUF

                @pl.when(j + 2 < n_chunks)
                def _launch_next_gather():
                    idx_copy(j + 2, bn).wait()

                    @pl.when(j >= 2)
                    def _wait_prev_write():
                        write_copy(j - 2, bn).wait()

                    gather_copy(bn).start()

                gather_copy(b).wait()

                @pl.when(j + _NBUF < n_chunks)
                def _stage_next_idx():
                    idx_copy(j + _NBUF, b).start()

                def row_body(r, c2):
                    for c in range(d_model // 16):
                        val = pos_v[pl.ds(r * d_model + c * 16, 16)]
                        plsc.addupdate(gb[b].at[r, 0, pl.ds(c * 16, 16)], val)
                    return c2

                lax.fori_loop(0, chunk, row_body, 0, unroll=2)

                write_copy(j, b).start()
            return carry

        lax.fori_loop(0, n_chunks // _NBUF, quad_body, 0)

        for j in range(n_chunks - _NBUF, n_chunks):
            write_copy(j, j % _NBUF).wait()

    out = _emb_kernel(flat_idx, table3, pos_flat)
    return out.reshape(batch, seq_len, d_model)
